# baseline (device time: 105132 ns/iter reference)
import jax
import jax.numpy as jnp
from jax import lax
from jax.experimental import pallas as pl
from jax.experimental.pallas import tpu as pltpu

B, H, D, BS = 8, 8, 64, 16
N_Y = 2


def kernel(Q, K, V, bt, lens):
    p_loc = K.shape[0]
    keys_loc = p_loc * BS
    keys_glob = N_Y * keys_loc

    Qt = Q.reshape(B, H, D).transpose(1, 0, 2)
    Kt = K.transpose(2, 0, 1, 3).reshape(H, keys_loc, D)
    Vt = V.transpose(2, 0, 1, 3).reshape(H, keys_loc, D)
    btT = bt.T

    def body(q_ref, k_ref, v_ref, btT_ref, lens_ref, out_ref,
             kfull, vfull, ckey_ref, send_sems, recv_sems, local_sems):
        my_x = lax.axis_index("x")
        my_y = lax.axis_index("y")
        my_z = lax.axis_index("z")
        nbr = (my_x, 1 - my_y, my_z)

        barrier_sem = pltpu.get_barrier_semaphore()
        pl.semaphore_signal(
            barrier_sem, inc=1, device_id=nbr,
            device_id_type=pl.DeviceIdType.MESH,
        )
        pl.semaphore_wait(barrier_sem, 1)

        copy_k = pltpu.make_async_copy(k_ref, kfull.at[my_y], local_sems.at[0])
        copy_v = pltpu.make_async_copy(v_ref, vfull.at[my_y], local_sems.at[1])
        copy_k.start()
        copy_v.start()
        rdma_k = pltpu.make_async_remote_copy(
            src_ref=k_ref, dst_ref=kfull.at[my_y],
            send_sem=send_sems.at[0], recv_sem=recv_sems.at[0],
            device_id=nbr, device_id_type=pl.DeviceIdType.MESH,
        )
        rdma_v = pltpu.make_async_remote_copy(
            src_ref=v_ref, dst_ref=vfull.at[my_y],
            send_sem=send_sems.at[1], recv_sem=recv_sems.at[1],
            device_id=nbr, device_id_type=pl.DeviceIdType.MESH,
        )
        rdma_k.start()
        rdma_v.start()

        btTv = btT_ref[...]
        page_of_key = lax.broadcasted_iota(jnp.int32, (p_loc, keys_glob), 1) // BS
        jrow = lax.broadcasted_iota(jnp.int32, (p_loc, keys_glob), 0)
        for i in range(B):
            li = lens_ref[i]
            btcol = btTv[:, i:i + 1]
            hit = (page_of_key == btcol) & (jrow < li)
            ckey_ref[i:i + 1, :] = jnp.sum(
                jnp.where(hit, 1.0, 0.0), axis=0, keepdims=True)

        copy_k.wait()
        copy_v.wait()
        rdma_k.wait()
        rdma_v.wait()

        ckey = ckey_ref[...]
        valid = ckey > 0.0
        qv = q_ref[...]
        scale = D ** -0.5
        for h in range(H):
            qh = qv[h]
            s0 = lax.dot_general(qh, kfull[0, h],
                                 (((1,), (1,)), ((), ())),
                                 preferred_element_type=jnp.float32)
            s1 = lax.dot_general(qh, kfull[1, h],
                                 (((1,), (1,)), ((), ())),
                                 preferred_element_type=jnp.float32)
            s = jnp.concatenate([s0, s1], axis=1) * scale
            s = jnp.where(valid, s, -1e30)
            m = jnp.max(s, axis=1, keepdims=True)
            e = ckey * jnp.exp(s - m)
            denom = jnp.sum(e, axis=1, keepdims=True)
            p = e / denom
            o = lax.dot_general(p[:, :keys_loc], vfull[0, h],
                                (((1,), (0,)), ((), ())),
                                preferred_element_type=jnp.float32)
            o = o + lax.dot_general(p[:, keys_loc:], vfull[1, h],
                                    (((1,), (0,)), ((), ())),
                                    preferred_element_type=jnp.float32)
            out_ref[h] = o

    out_t = pl.pallas_call(
        body,
        out_shape=jax.ShapeDtypeStruct((H, B, D), jnp.float32),
        in_specs=[
            pl.BlockSpec(memory_space=pltpu.VMEM),
            pl.BlockSpec(memory_space=pltpu.VMEM),
            pl.BlockSpec(memory_space=pltpu.VMEM),
            pl.BlockSpec(memory_space=pltpu.VMEM),
            pl.BlockSpec(memory_space=pltpu.SMEM),
        ],
        out_specs=pl.BlockSpec(memory_space=pltpu.VMEM),
        scratch_shapes=[
            pltpu.VMEM((N_Y, H, keys_loc, D), jnp.float32),
            pltpu.VMEM((N_Y, H, keys_loc, D), jnp.float32),
            pltpu.VMEM((B, keys_glob), jnp.float32),
            pltpu.SemaphoreType.DMA((2,)),
            pltpu.SemaphoreType.DMA((2,)),
            pltpu.SemaphoreType.DMA((2,)),
        ],
        compiler_params=pltpu.CompilerParams(collective_id=0),
    )(Qt, Kt, Vt, btT, lens)

    return out_t.transpose(1, 0, 2).reshape(B, 1, H, D)


# device time: 100741 ns/iter; 1.0436x vs baseline; 1.0436x over previous
import jax
import jax.numpy as jnp
from jax import lax
from jax.experimental import pallas as pl
from jax.experimental.pallas import tpu as pltpu

B, H, D, BS = 8, 8, 64, 16
N_Y = 2


def kernel(Q, K, V, bt, lens):
    p_loc = K.shape[0]
    keys_loc = p_loc * BS
    keys_glob = N_Y * keys_loc

    Qt = Q.reshape(B, H, D).transpose(1, 0, 2)
    Kt = K.transpose(2, 0, 1, 3).reshape(H, keys_loc, D)
    Vt = V.transpose(2, 0, 1, 3).reshape(H, keys_loc, D)
    btT = bt.T

    def body(q_ref, k_ref, v_ref, btT_ref, lens_ref, out_ref,
             krem, vrem, send_sems, recv_sems):
        my_x = lax.axis_index("x")
        my_y = lax.axis_index("y")
        my_z = lax.axis_index("z")
        nbr = (my_x, 1 - my_y, my_z)

        barrier_sem = pltpu.get_barrier_semaphore()
        pl.semaphore_signal(
            barrier_sem, inc=1, device_id=nbr,
            device_id_type=pl.DeviceIdType.MESH,
        )
        pl.semaphore_wait(barrier_sem, 1)

        rdma_k = pltpu.make_async_remote_copy(
            src_ref=k_ref, dst_ref=krem,
            send_sem=send_sems.at[0], recv_sem=recv_sems.at[0],
            device_id=nbr, device_id_type=pl.DeviceIdType.MESH,
        )
        rdma_v = pltpu.make_async_remote_copy(
            src_ref=v_ref, dst_ref=vrem,
            send_sem=send_sems.at[1], recv_sem=recv_sems.at[1],
            device_id=nbr, device_id_type=pl.DeviceIdType.MESH,
        )
        rdma_k.start()
        rdma_v.start()

        btTv = btT_ref[...]
        g = lax.broadcasted_iota(jnp.int32, (p_loc, keys_glob), 1)
        sign = jnp.where(g < keys_loc, 1, -1)
        page_of_key = g // BS + sign * my_y * p_loc
        jrow = lax.broadcasted_iota(jnp.int32, (p_loc, keys_glob), 0)
        counts = []
        for i in range(B):
            hit = (page_of_key == btTv[:, i:i + 1]) & (jrow < lens_ref[i])
            counts.append(jnp.sum(jnp.where(hit, 1.0, 0.0), axis=0,
                                  keepdims=True))
        ckey = jnp.concatenate(counts, axis=0)
        valid = ckey > 0.0

        qv = q_ref[...]
        scale = D ** -0.5
        s_loc = [
            lax.dot_general(qv[h], k_ref[h], (((1,), (1,)), ((), ())),
                            preferred_element_type=jnp.float32)
            for h in range(H)
        ]

        rdma_k.wait_recv()
        ps = []
        for h in range(H):
            s_rem = lax.dot_general(qv[h], krem[h], (((1,), (1,)), ((), ())),
                                    preferred_element_type=jnp.float32)
            s = jnp.concatenate([s_loc[h], s_rem], axis=1) * scale
            s = jnp.where(valid, s, -1e30)
            m = jnp.max(s, axis=1, keepdims=True)
            e = ckey * jnp.exp(s - m)
            ps.append(e / jnp.sum(e, axis=1, keepdims=True))

        rdma_v.wait_recv()
        for h in range(H):
            o = lax.dot_general(ps[h][:, :keys_loc], v_ref[h],
                                (((1,), (0,)), ((), ())),
                                preferred_element_type=jnp.float32)
            o = o + lax.dot_general(ps[h][:, keys_loc:], vrem[h],
                                    (((1,), (0,)), ((), ())),
                                    preferred_element_type=jnp.float32)
            out_ref[h] = o

        rdma_k.wait_send()
        rdma_v.wait_send()

    out_t = pl.pallas_call(
        body,
        out_shape=jax.ShapeDtypeStruct((H, B, D), jnp.float32),
        in_specs=[
            pl.BlockSpec(memory_space=pltpu.VMEM),
            pl.BlockSpec(memory_space=pltpu.VMEM),
            pl.BlockSpec(memory_space=pltpu.VMEM),
            pl.BlockSpec(memory_space=pltpu.VMEM),
            pl.BlockSpec(memory_space=pltpu.SMEM),
        ],
        out_specs=pl.BlockSpec(memory_space=pltpu.VMEM),
        scratch_shapes=[
            pltpu.VMEM((H, keys_loc, D), jnp.float32),
            pltpu.VMEM((H, keys_loc, D), jnp.float32),
            pltpu.SemaphoreType.DMA((2,)),
            pltpu.SemaphoreType.DMA((2,)),
        ],
        compiler_params=pltpu.CompilerParams(collective_id=0),
    )(Qt, Kt, Vt, btT, lens)

    return out_t.transpose(1, 0, 2).reshape(B, 1, H, D)


# device time: 31389 ns/iter; 3.3493x vs baseline; 3.2094x over previous
import jax
import jax.numpy as jnp
from jax import lax
from jax.experimental import pallas as pl
from jax.experimental.pallas import tpu as pltpu

B, H, D, BS = 8, 8, 64, 16
N_Y = 2
NEG = -1e30


def kernel(Q, K, V, bt, lens):
    p_loc = K.shape[0]
    keys_loc = p_loc * BS

    def body(q_ref, k_ref, v_ref, bt_ref, lens_ref, out_ref,
             u_scr, s_scr, urem, srem, send_sems, recv_sems):
        my_x = lax.axis_index("x")
        my_y = lax.axis_index("y")
        my_z = lax.axis_index("z")
        nbr = (my_x, 1 - my_y, my_z)

        barrier_sem = pltpu.get_barrier_semaphore()
        pl.semaphore_signal(
            barrier_sem, inc=1, device_id=nbr,
            device_id_type=pl.DeviceIdType.MESH,
        )

        k3 = k_ref[...].reshape(keys_loc, H, D)
        v3 = v_ref[...].reshape(keys_loc, H, D)
        btv = bt_ref[...]
        qs = q_ref[...] * (D ** -0.5)

        page_of_key = (
            lax.broadcasted_iota(jnp.int32, (keys_loc, p_loc), 0) // BS
            + my_y * p_loc
        )
        jcol = lax.broadcasted_iota(jnp.int32, (keys_loc, p_loc), 1)

        for i in range(B):
            hit = (page_of_key == btv[i:i + 1, :]) & (jcol < lens_ref[i])
            ckey = jnp.sum(jnp.where(hit, 1.0, 0.0), axis=1,
                           keepdims=True)
            s = jnp.sum(k3 * qs[i], axis=2)
            s = jnp.where(ckey > 0.0, s, NEG)
            m = jnp.max(s, axis=0, keepdims=True)
            e = ckey * jnp.exp(s - m)
            n = jnp.sum(e, axis=0, keepdims=True)
            u = jnp.sum(e[:, :, None] * v3, axis=0)
            u_scr[i] = u
            s_scr[0, i:i + 1, :] = m
            s_scr[1, i:i + 1, :] = n

        pl.semaphore_wait(barrier_sem, 1)
        rdma_u = pltpu.make_async_remote_copy(
            src_ref=u_scr, dst_ref=urem,
            send_sem=send_sems.at[0], recv_sem=recv_sems.at[0],
            device_id=nbr, device_id_type=pl.DeviceIdType.MESH,
        )
        rdma_s = pltpu.make_async_remote_copy(
            src_ref=s_scr, dst_ref=srem,
            send_sem=send_sems.at[1], recv_sem=recv_sems.at[1],
            device_id=nbr, device_id_type=pl.DeviceIdType.MESH,
        )
        rdma_u.start()
        rdma_s.start()
        rdma_u.wait()
        rdma_s.wait()

        m0, n0 = s_scr[0], s_scr[1]
        m1, n1 = srem[0], srem[1]
        mx = jnp.maximum(m0, m1)
        w0 = jnp.exp(m0 - mx)
        w1 = jnp.exp(m1 - mx)
        num = u_scr[...] * w0[:, :, None] + urem[...] * w1[:, :, None]
        den = n0 * w0 + n1 * w1
        out_ref[...] = (num / den[:, :, None])[:, None]

    return pl.pallas_call(
        body,
        out_shape=jax.ShapeDtypeStruct((B, 1, H, D), jnp.float32),
        in_specs=[
            pl.BlockSpec(memory_space=pltpu.VMEM),
            pl.BlockSpec(memory_space=pltpu.VMEM),
            pl.BlockSpec(memory_space=pltpu.VMEM),
            pl.BlockSpec(memory_space=pltpu.VMEM),
            pl.BlockSpec(memory_space=pltpu.SMEM),
        ],
        out_specs=pl.BlockSpec(memory_space=pltpu.VMEM),
        scratch_shapes=[
            pltpu.VMEM((B, H, D), jnp.float32),
            pltpu.VMEM((2, B, H), jnp.float32),
            pltpu.VMEM((B, H, D), jnp.float32),
            pltpu.VMEM((2, B, H), jnp.float32),
            pltpu.SemaphoreType.DMA((2,)),
            pltpu.SemaphoreType.DMA((2,)),
        ],
        compiler_params=pltpu.CompilerParams(collective_id=0),
    )(Q, K, V, bt, lens)


# device time: 19541 ns/iter; 5.3801x vs baseline; 1.6063x over previous
import jax
import jax.numpy as jnp
from jax import lax
from jax.experimental import pallas as pl
from jax.experimental.pallas import tpu as pltpu

B, H, D, BS = 8, 8, 64, 16
N_Y = 2
NEG = -1e30


def kernel(Q, K, V, bt, lens):
    p_loc = K.shape[0]
    keys_loc = p_loc * BS

    def body(q_ref, k_ref, v_ref, bt_ref, lens_ref, out_ref,
             u_scr, s_scr, urem, srem, send_sems, recv_sems):
        my_x = lax.axis_index("x")
        my_y = lax.axis_index("y")
        my_z = lax.axis_index("z")
        nbr = (my_x, 1 - my_y, my_z)

        barrier_sem = pltpu.get_barrier_semaphore()
        pl.semaphore_signal(
            barrier_sem, inc=1, device_id=nbr,
            device_id_type=pl.DeviceIdType.MESH,
        )

        btv = bt_ref[...]
        qsc = q_ref[...] * (D ** -0.5)

        page_of_key = (
            lax.broadcasted_iota(jnp.int32, (keys_loc, p_loc), 0) // BS
            + my_y * p_loc
        )
        jcol = lax.broadcasted_iota(jnp.int32, (keys_loc, p_loc), 1)
        cols = []
        for i in range(B):
            hit = (page_of_key == btv[i:i + 1, :]) & (jcol < lens_ref[i])
            cols.append(jnp.sum(jnp.where(hit, 1.0, 0.0), axis=1,
                                keepdims=True))
        ckey = jnp.concatenate(cols, axis=1)
        valid = ckey > 0.0

        for h in range(H):
            kh = k_ref[:, :, h, :].reshape(keys_loc, D)
            qh = qsc[:, 0, h, :]
            s = lax.dot_general(kh, qh, (((1,), (1,)), ((), ())),
                                preferred_element_type=jnp.float32)
            s = jnp.where(valid, s, NEG)
            m = jnp.max(s, axis=0, keepdims=True)
            e = ckey * jnp.exp(s - m)
            n = jnp.sum(e, axis=0, keepdims=True)
            vh = v_ref[:, :, h, :].reshape(keys_loc, D)
            u = lax.dot_general(e, vh, (((0,), (0,)), ((), ())),
                                preferred_element_type=jnp.float32)
            u_scr[h] = u
            s_scr[0, h:h + 1, :] = m
            s_scr[1, h:h + 1, :] = n

        pl.semaphore_wait(barrier_sem, 1)
        rdma_u = pltpu.make_async_remote_copy(
            src_ref=u_scr, dst_ref=urem,
            send_sem=send_sems.at[0], recv_sem=recv_sems.at[0],
            device_id=nbr, device_id_type=pl.DeviceIdType.MESH,
        )
        rdma_s = pltpu.make_async_remote_copy(
            src_ref=s_scr, dst_ref=srem,
            send_sem=send_sems.at[1], recv_sem=recv_sems.at[1],
            device_id=nbr, device_id_type=pl.DeviceIdType.MESH,
        )
        rdma_u.start()
        rdma_s.start()
        rdma_u.wait()
        rdma_s.wait()

        m0, n0 = s_scr[0], s_scr[1]
        m1, n1 = srem[0], srem[1]
        mx = jnp.maximum(m0, m1)
        w0 = jnp.exp(m0 - mx)
        w1 = jnp.exp(m1 - mx)
        num = u_scr[...] * w0[:, :, None] + urem[...] * w1[:, :, None]
        den = n0 * w0 + n1 * w1
        out_ref[...] = num / den[:, :, None]

    out_t = pl.pallas_call(
        body,
        out_shape=jax.ShapeDtypeStruct((H, B, D), jnp.float32),
        in_specs=[
            pl.BlockSpec(memory_space=pltpu.VMEM),
            pl.BlockSpec(memory_space=pltpu.VMEM),
            pl.BlockSpec(memory_space=pltpu.VMEM),
            pl.BlockSpec(memory_space=pltpu.VMEM),
            pl.BlockSpec(memory_space=pltpu.SMEM),
        ],
        out_specs=pl.BlockSpec(memory_space=pltpu.VMEM),
        scratch_shapes=[
            pltpu.VMEM((H, B, D), jnp.float32),
            pltpu.VMEM((2, H, B), jnp.float32),
            pltpu.VMEM((H, B, D), jnp.float32),
            pltpu.VMEM((2, H, B), jnp.float32),
            pltpu.SemaphoreType.DMA((2,)),
            pltpu.SemaphoreType.DMA((2,)),
        ],
        compiler_params=pltpu.CompilerParams(collective_id=0),
    )(Q, K, V, bt, lens)

    return out_t.transpose(1, 0, 2).reshape(B, 1, H, D)
